# baseline (speedup 1.0000x reference)
import jax
import jax.numpy as jnp
from jax.experimental import pallas as pl
from jax.experimental.pallas import tpu as pltpu

_B, _C, _H, _W, _K = 8, 80, 64, 64, 128
_HW = _H * _W
_ROWS = _B * _C
_BLK = 64
_GRID = _ROWS // _BLK
_EPS = 0.0001


def _focal_part(x, gt, valid):
    p = jnp.clip(jax.nn.sigmoid(x), _EPS, 1.0 - _EPS)
    posf = gt == 1.0
    negf = (gt < 1.0).astype(jnp.float32)
    t = jnp.where(posf, p, 1.0 - p)
    l = jnp.log(t)
    neg_w = (1.0 - gt) ** 4
    f = jnp.where(posf, (1.0 - p) ** 2, p * p * neg_w * negf)
    v = l * f
    if valid is not None:
        v = v * valid
    return jnp.sum(v), jnp.sum(posf.astype(jnp.float32))


def _gather_cols(feat_ref, off_ref, ind_ref):
    iota = jax.lax.broadcasted_iota(jnp.int32, (_K, _W), 1)
    tcols, ocols = [], []
    for b in range(_B):
        ind = ind_ref[:, b : b + 1]
        hi = (ind // _W) == iota
        lo = ((ind % _W) == iota).astype(jnp.float32)
        ohh = hi.astype(jnp.float32)
        img = feat_ref[_H * b : _H * (b + 1), :]
        g1 = jnp.dot(ohh, img, preferred_element_type=jnp.float32)
        tcols.append(jnp.sum(g1 * lo, axis=1, keepdims=True))
        base = 2 * _H * b
        for c in range(2):
            oimg = off_ref[base + _H * c : base + _H * (c + 1), :]
            g2 = jnp.dot(ohh, oimg, preferred_element_type=jnp.float32)
            ocols.append(jnp.sum(g2 * lo, axis=1, keepdims=True))
    return jnp.concatenate(tcols, axis=1), jnp.concatenate(ocols, axis=1)


def _body(tlx, brx, gtl, gbr, vtl, vbr,
          a0, a1, a2, ga0, ga1, ga2,
          tagtl, tagbr, offtl, offbr,
          indtl, indbr, maskt, mask2t, gofftl, goffbr,
          out_ref, acc):
    i = pl.program_id(0)

    @pl.when(i == 0)
    def _init():
        att_total = 0.0
        for a_ref, g_ref in ((a0, ga0), (a1, ga1), (a2, ga2)):
            s, npos = _focal_part(a_ref[...], g_ref[...], None)
            att_total += -s / npos

        t0, o_tl = _gather_cols(tagtl, offtl, indtl)
        t1, o_br = _gather_cols(tagbr, offbr, indbr)
        m = maskt[...]
        num = jnp.sum(m, axis=0, keepdims=True)
        mean = (t0 + t1) * 0.5
        pull = (jnp.sum((t0 - mean) ** 2 / (num + _EPS) * m)
                + jnp.sum((t1 - mean) ** 2 / (num + _EPS) * m))

        m2 = mask2t[...]
        numtot = jnp.sum(m)

        def huber_sum(o, goff):
            d = o - goff
            ad = jnp.abs(d)
            return jnp.sum(jnp.where(ad < 1.0, 0.5 * d * d, ad - 0.5) * m2)

        off_total = (huber_sum(o_tl, gofftl[...])
                     + huber_sum(o_br, goffbr[...])) / (numtot + _EPS)

        acc[0] = 0.0
        acc[1] = 0.0
        acc[2] = 0.0
        acc[3] = 0.0
        acc[4] = att_total + pull + off_total

    s_tl, np_tl = _focal_part(tlx[...], gtl[...], vtl[...])
    s_br, np_br = _focal_part(brx[...], gbr[...], vbr[...])
    acc[0] += s_tl
    acc[1] += np_tl
    acc[2] += s_br
    acc[3] += np_br

    @pl.when(i == _GRID - 1)
    def _fin():
        out_ref[0, 0] = -acc[0] / acc[1] - acc[2] / acc[3] + acc[4]


def _run(args, interpret=False):
    big = pl.BlockSpec((_BLK, _HW), lambda i: (i, 0))

    def full(shape):
        return pl.BlockSpec(shape, lambda i: (0,) * len(shape))

    small_shapes = [
        (_B, 256), (_B, 1024), (_B, _HW),
        (_B, 256), (_B, 1024), (_B, _HW),
        (_B * _H, _W), (_B * _H, _W),
        (_B * 2 * _H, _W), (_B * 2 * _H, _W),
        (_K, _B), (_K, _B),
        (_K, _B), (_K, 2 * _B),
        (_K, 2 * _B), (_K, 2 * _B),
    ]
    out = pl.pallas_call(
        _body,
        grid=(_GRID,),
        in_specs=[big] * 6 + [full(s) for s in small_shapes],
        out_specs=pl.BlockSpec(memory_space=pltpu.SMEM),
        out_shape=jax.ShapeDtypeStruct((1, 1), jnp.float32),
        scratch_shapes=[pltpu.SMEM((8,), jnp.float32)],
        interpret=interpret,
    )(*args)
    return out.reshape(1)


def kernel(tl_heat, br_heat, tl_tag, br_tag, tl_off, br_off, att0, att1,
           att2, gt_tl_heat, gt_br_heat, gt_mask, gt_tl_off, gt_br_off,
           gt_tl_ind, gt_br_ind, gt_tl_valid, gt_br_valid, gt_att0,
           gt_att1, gt_att2, *, _interpret=False):
    f32 = jnp.float32
    args = (
        tl_heat.reshape(_ROWS, _HW), br_heat.reshape(_ROWS, _HW),
        gt_tl_heat.reshape(_ROWS, _HW), gt_br_heat.reshape(_ROWS, _HW),
        gt_tl_valid.reshape(_ROWS, _HW), gt_br_valid.reshape(_ROWS, _HW),
        att0.reshape(_B, 256), att1.reshape(_B, 1024), att2.reshape(_B, _HW),
        gt_att0.reshape(_B, 256), gt_att1.reshape(_B, 1024),
        gt_att2.reshape(_B, _HW),
        tl_tag.reshape(_B * _H, _W), br_tag.reshape(_B * _H, _W),
        tl_off.reshape(_B * 2 * _H, _W), br_off.reshape(_B * 2 * _H, _W),
        gt_tl_ind.astype(jnp.int32).T, gt_br_ind.astype(jnp.int32).T,
        gt_mask.astype(f32).T,
        jnp.repeat(gt_mask.astype(f32).T, 2, axis=1),
        jnp.transpose(gt_tl_off, (1, 0, 2)).reshape(_K, 2 * _B),
        jnp.transpose(gt_br_off, (1, 0, 2)).reshape(_K, 2 * _B),
    )
    return _run(args, interpret=_interpret)
